# baseline (speedup 1.0000x reference)
import jax
import jax.numpy as jnp
from jax.experimental import pallas as pl
import jax.experimental.pallas.tpu as pltpu

K_BINS = 1024
WIDTH = 64
TT = 4096


def _dist_tile(x_ref, cb_ref, csq_ref):
    xt = x_ref[0]
    cb = cb_ref[...]
    csq = csq_ref[...]
    xsq = jnp.sum(xt * xt, axis=0, keepdims=True)
    mm2 = jax.lax.dot_general(
        cb, -2.0 * xt, (((1,), (0,)), ((), ())),
        preferred_element_type=jnp.float32,
    )
    dist = (xsq + mm2) + csq
    return xt, xsq, dist


def _vq_fast_kernel(x_ref, cb_ref, cbe_ref, csq_ref,
                    xl_ref, xd_ref, cnt_ref, fit_ref, sum_ref, sq_ref):
    cbe = cbe_ref[...]
    xt, xsq, dist = _dist_tile(x_ref, cb_ref, csq_ref)

    minval = jnp.min(dist, axis=0, keepdims=True)
    onehot = (dist == minval).astype(jnp.bfloat16)

    res = jax.lax.dot_general(
        cbe, onehot, (((0,), (0,)), ((), ())),
        preferred_element_type=jnp.float32,
    )
    idx = (res[WIDTH] + res[WIDTH + 1]).astype(jnp.int32)

    xl_ref[0, 0, :] = idx
    xd_ref[0] = res[:WIDTH]
    cnt_ref[0, 0, :] = res[WIDTH + 2]

    fit_ref[...] = jnp.sum(minval).reshape(1, 1, 1)
    sum_ref[...] = jnp.sum(xt).reshape(1, 1, 1)
    sq_ref[...] = jnp.sum(xsq).reshape(1, 1, 1)


def _vq_exact_kernel(x_ref, cb_ref, cbe_ref, csq_ref, xl_ref, xd_ref):
    cbe = cbe_ref[...]
    _, _, dist = _dist_tile(x_ref, cb_ref, csq_ref)
    minval = jnp.min(dist, axis=0, keepdims=True)
    kiota = jax.lax.broadcasted_iota(jnp.int32, dist.shape, 0)
    idx = jnp.min(
        jnp.where(dist == minval, kiota, K_BINS), axis=0
    )
    onehot = (kiota == idx[None, :]).astype(jnp.bfloat16)
    xd = jax.lax.dot_general(
        cbe[:, :WIDTH], onehot, (((0,), (0,)), ((), ())),
        preferred_element_type=jnp.float32,
    )
    xl_ref[0, 0, :] = idx
    xd_ref[0] = xd


def kernel(x, codebook):
    N, width, T = x.shape
    G = T // TT
    numel = float(N * T * width)

    ones_k = jnp.ones((K_BINS, 1), jnp.float32)
    k_int = jnp.arange(K_BINS, dtype=jnp.int32)[:, None]
    k_hi = (k_int & ~3).astype(jnp.float32)
    k_lo = (k_int & 3).astype(jnp.float32)
    cb_ext = jnp.concatenate(
        [codebook, k_hi, k_lo, ones_k], axis=1
    ).astype(jnp.bfloat16)
    csq_col = jnp.sum(codebook.T ** 2, axis=0)[:, None]

    grid = (N, G)
    in_specs = [
        pl.BlockSpec((1, width, TT), lambda i, j: (i, 0, j)),
        pl.BlockSpec((K_BINS, width), lambda i, j: (0, 0)),
        pl.BlockSpec((K_BINS, width + 3), lambda i, j: (0, 0)),
        pl.BlockSpec((K_BINS, 1), lambda i, j: (0, 0)),
    ]
    xl_spec = pl.BlockSpec((1, 1, TT), lambda i, j: (i * G + j, 0, 0))
    xd_spec = pl.BlockSpec((1, width, TT), lambda i, j: (i, 0, j))
    part_spec = pl.BlockSpec((1, 1, 1), lambda i, j: (i * G + j, 0, 0))
    xl_shape = jax.ShapeDtypeStruct((N * G, 1, TT), jnp.int32)
    xd_shape = jax.ShapeDtypeStruct((N, width, T), jnp.float32)
    part_shape = jax.ShapeDtypeStruct((N * G, 1, 1), jnp.float32)
    cparams = pltpu.CompilerParams(
        dimension_semantics=(pltpu.PARALLEL, pltpu.PARALLEL),
    )

    xl3, x_d, cnt3, fit_p, sum_p, sq_p = pl.pallas_call(
        _vq_fast_kernel,
        grid=grid,
        in_specs=in_specs,
        out_specs=(xl_spec, xd_spec,
                   pl.BlockSpec((1, 1, TT), lambda i, j: (i * G + j, 0, 0)),
                   part_spec, part_spec, part_spec),
        out_shape=(xl_shape, xd_shape,
                   jax.ShapeDtypeStruct((N * G, 1, TT), jnp.float32),
                   part_shape, part_shape, part_shape),
        compiler_params=cparams,
    )(x, codebook, cb_ext, csq_col)

    def _exact(_):
        return pl.pallas_call(
            _vq_exact_kernel,
            grid=grid,
            in_specs=in_specs,
            out_specs=(xl_spec, xd_spec),
            out_shape=(xl_shape, xd_shape),
            compiler_params=cparams,
        )(x, codebook, cb_ext, csq_col)

    def _fast(_):
        return xl3, x_d

    xl3, x_d = jax.lax.cond(jnp.max(cnt3) > 1.5, _exact, _fast, None)

    x_l = xl3.reshape(N, T)
    fit_sum = jnp.sum(fit_p)
    s = jnp.sum(sum_p)
    sq = jnp.sum(sq_p)

    fit = fit_sum / (N * T)
    commit_loss = fit_sum / numel
    mean = s / numel
    prenorm = jnp.sqrt(jnp.maximum(sq / numel - mean * mean, 0.0))
    return x_d, commit_loss, fit, prenorm, x_l
